# baseline (speedup 1.0000x reference)
import functools

import jax
import jax.numpy as jnp
from jax import lax
from jax.experimental import pallas as pl
from jax.experimental.pallas import tpu as pltpu
from jax.experimental.pallas import tpu_sc as plsc


def _sc_gather(table, idx):
    B, = idx.shape
    V, D = table.shape
    info = plsc.get_sparse_core_info()
    NC, NS = info.num_cores, info.num_subcores
    NW = NC * NS
    b_per_w = B // NW
    ch = b_per_w
    while ch * D * 4 > 262144:
        ch //= 2
    n_ch = b_per_w // ch

    mesh = plsc.VectorSubcoreMesh(core_axis_name="c", subcore_axis_name="s")

    @functools.partial(
        pl.kernel,
        mesh=mesh,
        out_type=jax.ShapeDtypeStruct((B, D), jnp.float32),
        scratch_types=[
            pltpu.VMEM((ch,), jnp.int32),
            pltpu.VMEM((ch, D), jnp.float32),
            pltpu.SemaphoreType.DMA,
        ],
    )
    def gather_kernel(table_hbm, idx_hbm, out_hbm, idx_v, rows_v, sem):
        wid = lax.axis_index("s") * NC + lax.axis_index("c")
        base = wid * b_per_w
        for c in range(n_ch):
            off = base + c * ch
            pltpu.sync_copy(idx_hbm.at[pl.ds(off, ch)], idx_v)
            pltpu.async_copy(table_hbm.at[idx_v], rows_v, sem).wait()
            pltpu.sync_copy(rows_v, out_hbm.at[pl.ds(off, ch)])

    return gather_kernel(table, idx)


def _moe_ffn(h_bf, ids, rw, W1, b1, W2, b2):
    S, D = h_bf.shape
    E, _, FF = W1.shape
    K = ids.shape[0]
    Fb = min(512, FF)
    NF = FF // Fb

    def body(ids_ref, rw_ref, h_ref, w1_ref, b1_ref, w2_ref, b2_ref, out_ref):
        k = pl.program_id(0)
        f = pl.program_id(1)
        w = rw_ref[k]
        w1 = w1_ref[0].astype(jnp.bfloat16)
        inter = jnp.dot(h_ref[...], w1, preferred_element_type=jnp.float32)
        inter = (jax.nn.gelu(inter + b1_ref[0]) * w).astype(jnp.bfloat16)
        w2 = w2_ref[0].astype(jnp.bfloat16)
        contrib = jnp.dot(inter, w2, preferred_element_type=jnp.float32)

        @pl.when((k == 0) & (f == 0))
        def _():
            out_ref[...] = contrib + w * b2_ref[0]

        @pl.when((k > 0) & (f == 0))
        def _():
            out_ref[...] += contrib + w * b2_ref[0]

        @pl.when(f > 0)
        def _():
            out_ref[...] += contrib

    grid_spec = pltpu.PrefetchScalarGridSpec(
        num_scalar_prefetch=2,
        grid=(K, NF),
        in_specs=[
            pl.BlockSpec((S, D), lambda k, f, ids, rw: (0, 0)),
            pl.BlockSpec((1, D, Fb), lambda k, f, ids, rw: (ids[k], 0, f)),
            pl.BlockSpec((1, 1, Fb), lambda k, f, ids, rw: (ids[k], 0, f)),
            pl.BlockSpec((1, Fb, D), lambda k, f, ids, rw: (ids[k], f, 0)),
            pl.BlockSpec((1, 1, D), lambda k, f, ids, rw: (ids[k], 0, 0)),
        ],
        out_specs=pl.BlockSpec((S, D), lambda k, f, ids, rw: (0, 0)),
    )
    return pl.pallas_call(
        body,
        grid_spec=grid_spec,
        out_shape=jax.ShapeDtypeStruct((S, D), jnp.float32),
        compiler_params=pltpu.CompilerParams(
            dimension_semantics=("arbitrary", "arbitrary"),
            vmem_limit_bytes=112 * 1024 * 1024,
        ),
    )(ids, rw, h_bf, W1, b1.reshape(E, 1, FF), W2, b2.reshape(E, 1, D))


def _head_body(c_ref, t_ref, o_ref):
    t = t_ref[...].astype(jnp.bfloat16)
    o_ref[...] = lax.dot_general(
        c_ref[...], t, (((1,), (1,)), ((), ())),
        preferred_element_type=jnp.float32)


def _head(combined_bf, tok_emb):
    S, D = combined_bf.shape
    V, _ = tok_emb.shape
    Vb = 640 if V % 640 == 0 else V
    NV = V // Vb
    return pl.pallas_call(
        _head_body,
        grid=(NV,),
        in_specs=[
            pl.BlockSpec((S, D), lambda v: (0, 0)),
            pl.BlockSpec((Vb, D), lambda v: (v, 0)),
        ],
        out_specs=pl.BlockSpec((S, Vb), lambda v: (0, v)),
        out_shape=jax.ShapeDtypeStruct((S, V), jnp.float32),
        compiler_params=pltpu.CompilerParams(
            dimension_semantics=("arbitrary",),
            vmem_limit_bytes=112 * 1024 * 1024,
        ),
    )(combined_bf, tok_emb)


def kernel(input_ids, tok_emb, pos_emb, Wr, W1, b1, W2, b2):
    Bsz, S = input_ids.shape
    V, D = tok_emb.shape
    TOPK = 2

    idx = input_ids.reshape(-1).astype(jnp.int32)
    gathered = _sc_gather(tok_emb, idx)
    h = gathered + jnp.tile(pos_emb[:S], (Bsz, 1))

    hB = h.reshape(Bsz, S, D)
    pooled = hB.mean(axis=(0, 1))
    scores = Wr @ pooled
    vals, ids = lax.top_k(scores, TOPK)
    rw = jax.nn.softmax(vals)

    h_bf = h.astype(jnp.bfloat16)
    combined = _moe_ffn(h_bf, ids.astype(jnp.int32), rw, W1, b1, W2, b2)
    logits = _head(combined.astype(jnp.bfloat16), tok_emb)
    return logits.reshape(Bsz, S, V)


# speedup vs baseline: 1.0824x; 1.0824x over previous
import functools

import jax
import jax.numpy as jnp
from jax import lax
from jax.experimental import pallas as pl
from jax.experimental.pallas import tpu as pltpu
from jax.experimental.pallas import tpu_sc as plsc


def _sc_gather(table, idx):
    B, = idx.shape
    V, D = table.shape
    info = plsc.get_sparse_core_info()
    NC, NS = info.num_cores, info.num_subcores
    NW = NC * NS
    b_per_w = B // NW
    ch = b_per_w
    while ch * D * 4 > 262144:
        ch //= 2
    n_ch = b_per_w // ch

    mesh = plsc.VectorSubcoreMesh(core_axis_name="c", subcore_axis_name="s")

    @functools.partial(
        pl.kernel,
        mesh=mesh,
        out_type=jax.ShapeDtypeStruct((B, D), jnp.float32),
        scratch_types=[
            pltpu.VMEM((ch,), jnp.int32),
            pltpu.VMEM((ch, D), jnp.float32),
            pltpu.SemaphoreType.DMA,
        ],
    )
    def gather_kernel(table_hbm, idx_hbm, out_hbm, idx_v, rows_v, sem):
        wid = lax.axis_index("s") * NC + lax.axis_index("c")
        base = wid * b_per_w
        for c in range(n_ch):
            off = base + c * ch
            pltpu.sync_copy(idx_hbm.at[pl.ds(off, ch)], idx_v)
            pltpu.async_copy(table_hbm.at[idx_v], rows_v, sem).wait()
            pltpu.sync_copy(rows_v, out_hbm.at[pl.ds(off, ch)])

    return gather_kernel(table, idx)


def _moe_ffn(h_bf, ids, rw, W1, b1, W2, b2):
    S, D = h_bf.shape
    E, _, FF = W1.shape
    K = ids.shape[0]
    Fb = min(512, FF)
    NF = FF // Fb

    def body(ids_ref, rw_ref, h_ref, w1_ref, b1_ref, w2_ref, b2_ref, out_ref):
        k = pl.program_id(0)
        f = pl.program_id(1)
        w = rw_ref[k]
        w1 = w1_ref[0].astype(jnp.bfloat16)
        inter = jnp.dot(h_ref[...], w1, preferred_element_type=jnp.float32)
        inter = (jax.nn.gelu(inter + b1_ref[0]) * w).astype(jnp.bfloat16)
        w2 = w2_ref[0].astype(jnp.bfloat16)
        contrib = jnp.dot(inter, w2, preferred_element_type=jnp.float32)

        @pl.when((k == 0) & (f == 0))
        def _():
            out_ref[...] = contrib + w * b2_ref[0]

        @pl.when((k > 0) & (f == 0))
        def _():
            out_ref[...] += contrib + w * b2_ref[0]

        @pl.when(f > 0)
        def _():
            out_ref[...] += contrib

    grid_spec = pltpu.PrefetchScalarGridSpec(
        num_scalar_prefetch=2,
        grid=(K, NF),
        in_specs=[
            pl.BlockSpec((S, D), lambda k, f, ids, rw: (0, 0)),
            pl.BlockSpec((1, D, Fb), lambda k, f, ids, rw: (ids[k], 0, f)),
            pl.BlockSpec((1, 1, Fb), lambda k, f, ids, rw: (ids[k], 0, f)),
            pl.BlockSpec((1, Fb, D), lambda k, f, ids, rw: (ids[k], f, 0)),
            pl.BlockSpec((1, 1, D), lambda k, f, ids, rw: (ids[k], 0, 0)),
        ],
        out_specs=pl.BlockSpec((S, D), lambda k, f, ids, rw: (0, 0)),
    )
    return pl.pallas_call(
        body,
        grid_spec=grid_spec,
        out_shape=jax.ShapeDtypeStruct((S, D), jnp.float32),
        compiler_params=pltpu.CompilerParams(
            dimension_semantics=("arbitrary", "arbitrary"),
            vmem_limit_bytes=112 * 1024 * 1024,
        ),
    )(ids, rw, h_bf, W1, b1.reshape(E, 1, FF), W2, b2.reshape(E, 1, D))


def _head_body(c_ref, t_ref, o_ref):
    t = t_ref[...].astype(jnp.bfloat16)
    o_ref[...] = lax.dot_general(
        c_ref[...], t, (((1,), (1,)), ((), ())),
        preferred_element_type=jnp.float32)


def _head(combined_bf, tok_emb):
    S, D = combined_bf.shape
    V, _ = tok_emb.shape
    Vb = 1280 if V % 1280 == 0 else V
    NV = V // Vb
    return pl.pallas_call(
        _head_body,
        grid=(NV,),
        in_specs=[
            pl.BlockSpec((S, D), lambda v: (0, 0)),
            pl.BlockSpec((Vb, D), lambda v: (v, 0)),
        ],
        out_specs=pl.BlockSpec((S, Vb), lambda v: (0, v)),
        out_shape=jax.ShapeDtypeStruct((S, V), jnp.float32),
        compiler_params=pltpu.CompilerParams(
            dimension_semantics=("arbitrary",),
            vmem_limit_bytes=112 * 1024 * 1024,
        ),
    )(combined_bf, tok_emb)


def kernel(input_ids, tok_emb, pos_emb, Wr, W1, b1, W2, b2):
    Bsz, S = input_ids.shape
    V, D = tok_emb.shape
    TOPK = 2

    idx = input_ids.reshape(-1).astype(jnp.int32)
    gathered = _sc_gather(tok_emb, idx)
    h = gathered + jnp.tile(pos_emb[:S], (Bsz, 1))

    hB = h.reshape(Bsz, S, D)
    pooled = hB.mean(axis=(0, 1))
    scores = Wr @ pooled
    vals, ids = lax.top_k(scores, TOPK)
    rw = jax.nn.softmax(vals)

    h_bf = h.astype(jnp.bfloat16)
    combined = _moe_ffn(h_bf, ids.astype(jnp.int32), rw, W1, b1, W2, b2)
    logits = _head(combined.astype(jnp.bfloat16), tok_emb)
    return logits.reshape(Bsz, S, V)
